# Initial kernel scaffold; baseline (speedup 1.0000x reference)
import jax
import jax.numpy as jnp
from jax.experimental import pallas as pl


def kernel(batch_box_preds_tea, batch_box_preds_stu, batch_cls_preds_stu, cls_preds):
    raise NotImplementedError("write your pallas kernel here")


# speedup vs baseline: 45.0677x; 45.0677x over previous
import functools

import jax
import jax.numpy as jnp
from jax.experimental import pallas as pl
from jax.experimental.pallas import tpu as pltpu

_B, _N, _C = 4, 20000, 3
_PRE = 2048
_POST = 500
_POST_PAD = 512
_IOU_TH = 0.7
_SCORE_TH = 0.1
_STRIP = 256


def _nms_body(a_ref, bt_ref, tc_ref, sel_ref, small_ref, s_scr):
    A = a_ref[0]
    Bt = bt_ref[0]
    TC = tc_ref[0]

    f32 = jnp.float32

    x1c = A[:, 0:1] - A[:, 3:4] * 0.5
    x2c = A[:, 0:1] + A[:, 3:4] * 0.5
    y1c = A[:, 1:2] - A[:, 4:5] * 0.5
    y2c = A[:, 1:2] + A[:, 4:5] * 0.5
    areac = (x2c - x1c) * (y2c - y1c)

    x1r = Bt[0:1, :] - Bt[3:4, :] * 0.5
    x2r = Bt[0:1, :] + Bt[3:4, :] * 0.5
    y1r = Bt[1:2, :] - Bt[4:5, :] * 0.5
    y2r = Bt[1:2, :] + Bt[4:5, :] * 0.5
    arear = (x2r - x1r) * (y2r - y1r)

    ts_row = Bt[7:8, :]
    valid = ts_row > -jnp.inf

    for k in range(_PRE // _STRIP):
        r0 = k * _STRIP
        xc1 = x1c[r0:r0 + _STRIP]
        xc2 = x2c[r0:r0 + _STRIP]
        yc1 = y1c[r0:r0 + _STRIP]
        yc2 = y2c[r0:r0 + _STRIP]
        ac = areac[r0:r0 + _STRIP]
        iw = jnp.maximum(
            jnp.minimum(xc2, x2r) - jnp.maximum(xc1, x1r), 0.0)
        ih = jnp.maximum(
            jnp.minimum(yc2, y2r) - jnp.maximum(yc1, y1r), 0.0)
        inter = iw * ih
        iou = inter / (ac + arear - inter + 1e-6)
        ii = jax.lax.broadcasted_iota(jnp.int32, (_STRIP, _PRE), 0) + r0
        jj = jax.lax.broadcasted_iota(jnp.int32, (_STRIP, _PRE), 1)
        s_scr[r0:r0 + _STRIP, :] = jnp.where(
            (iou > _IOU_TH) & (jj > ii), 1.0, 0.0).astype(f32)

    S = s_scr[...]

    dot = functools.partial(
        jnp.dot, preferred_element_type=f32,
        precision=jax.lax.Precision.HIGHEST)

    valid_f = valid.astype(f32)

    def cond(st):
        _, _, undecided = st
        return undecided > 0.0

    def step(st):
        kept, dead, _ = st
        alive = valid_f * (1.0 - dead)
        threat = dot(alive, S)
        kept_n = jnp.where(threat < 0.5, valid_f, 0.0)
        dead_n = jnp.where(dot(kept_n, S) > 0.5, 1.0, 0.0)
        und = jnp.sum(valid_f * (1.0 - kept_n) * (1.0 - dead_n))
        return kept_n, dead_n, und

    zeros_f = jnp.zeros((1, _PRE), dtype=f32)
    und0 = jnp.sum(valid_f)
    kept_f, _, _ = jax.lax.while_loop(
        cond, step, (zeros_f, zeros_f, und0))

    for k in range(_PRE // _STRIP):
        r0 = k * _STRIP
        ii = jax.lax.broadcasted_iota(jnp.int32, (_STRIP, _PRE), 0) + r0
        jj = jax.lax.broadcasted_iota(jnp.int32, (_STRIP, _PRE), 1)
        s_scr[r0:r0 + _STRIP, :] = jnp.where(ii <= jj, 1.0, 0.0)
    rank = dot(kept_f, s_scr[...])
    slot = rank - 1.0

    rows = jax.lax.broadcasted_iota(jnp.int32, (_POST_PAD, 1), 0)
    slot_i = slot.astype(jnp.int32)
    O = jnp.where((rows == slot_i) & (kept_f > 0.5), 1.0, 0.0).astype(f32)

    sel = dot(O, TC)
    sel_ref[0] = sel

    kcnt = jnp.sum(kept_f).astype(jnp.int32)
    maskc = jnp.where(rows < kcnt, 1.0, 0.0).astype(f32)
    cmax = jnp.max(sel[:, 8:11], axis=1, keepdims=True)
    small = jnp.concatenate(
        [maskc, cmax * maskc, jnp.zeros((_POST_PAD, 6), f32)], axis=1)
    small_ref[0] = small


def kernel(batch_box_preds_tea, batch_box_preds_stu, batch_cls_preds_stu,
           cls_preds):
    f32 = jnp.float32
    dense_cls = cls_preds.reshape(_B, -1, _C)

    scores = jnp.max(batch_cls_preds_stu, axis=2)
    masked = jnp.where(scores > _SCORE_TH, scores, -jnp.inf)
    top_scores, order = jax.lax.top_k(masked, _PRE)

    oix = order[:, :, None]
    cand_stu = jnp.take_along_axis(batch_box_preds_stu, oix, axis=1)
    cand_tea = jnp.take_along_axis(batch_box_preds_tea, oix, axis=1)
    cand_cls = jnp.take_along_axis(dense_cls, oix, axis=1)

    a_in = jnp.concatenate([cand_stu, top_scores[:, :, None]], axis=2)
    bt_in = jnp.swapaxes(a_in, 1, 2)
    tc_in = jnp.concatenate(
        [cand_tea, jnp.zeros((_B, _PRE, 1), f32),
         cand_cls, jnp.zeros((_B, _PRE, 5), f32)], axis=2)

    sel, small = pl.pallas_call(
        _nms_body,
        grid=(_B,),
        in_specs=[
            pl.BlockSpec((1, _PRE, 8), lambda b: (b, 0, 0)),
            pl.BlockSpec((1, 8, _PRE), lambda b: (b, 0, 0)),
            pl.BlockSpec((1, _PRE, 16), lambda b: (b, 0, 0)),
        ],
        out_specs=[
            pl.BlockSpec((1, _POST_PAD, 16), lambda b: (b, 0, 0)),
            pl.BlockSpec((1, _POST_PAD, 8), lambda b: (b, 0, 0)),
        ],
        out_shape=[
            jax.ShapeDtypeStruct((_B, _POST_PAD, 16), f32),
            jax.ShapeDtypeStruct((_B, _POST_PAD, 8), f32),
        ],
        scratch_shapes=[pltpu.VMEM((_PRE, _PRE), f32)],
    )(a_in, bt_in, tc_in)

    rois = sel[:, :_POST, 0:7]
    cls_sel = sel[:, :_POST, 8:11]
    mask = small[:, :_POST, 0]
    kd = small[:, :_POST, 1].reshape(-1, 1)
    return rois, cls_sel, mask, kd


# speedup vs baseline: 58.0217x; 1.2874x over previous
import functools

import jax
import jax.numpy as jnp
from jax.experimental import pallas as pl
from jax.experimental.pallas import tpu as pltpu

_B, _N, _C = 4, 20000, 3
_PRE = 2048
_POST = 500
_POST_PAD = 512
_IOU_TH = 0.7
_SCORE_TH = 0.1
_STRIP = 256


def _nms_body(a_ref, bt_ref, tc_ref, sel_ref, small_ref, s_scr):
    A = a_ref[0]
    Bt = bt_ref[0]
    TC = tc_ref[0]

    f32 = jnp.float32

    x1c = A[:, 0:1] - A[:, 3:4] * 0.5
    x2c = A[:, 0:1] + A[:, 3:4] * 0.5
    y1c = A[:, 1:2] - A[:, 4:5] * 0.5
    y2c = A[:, 1:2] + A[:, 4:5] * 0.5
    areac = (x2c - x1c) * (y2c - y1c)

    x1r = Bt[0:1, :] - Bt[3:4, :] * 0.5
    x2r = Bt[0:1, :] + Bt[3:4, :] * 0.5
    y1r = Bt[1:2, :] - Bt[4:5, :] * 0.5
    y2r = Bt[1:2, :] + Bt[4:5, :] * 0.5
    arear = (x2r - x1r) * (y2r - y1r)

    ts_row = Bt[7:8, :]
    valid = ts_row > -jnp.inf

    for k in range(_PRE // _STRIP):
        r0 = k * _STRIP
        xc1 = x1c[r0:r0 + _STRIP]
        xc2 = x2c[r0:r0 + _STRIP]
        yc1 = y1c[r0:r0 + _STRIP]
        yc2 = y2c[r0:r0 + _STRIP]
        ac = areac[r0:r0 + _STRIP]
        iw = jnp.maximum(
            jnp.minimum(xc2, x2r) - jnp.maximum(xc1, x1r), 0.0)
        ih = jnp.maximum(
            jnp.minimum(yc2, y2r) - jnp.maximum(yc1, y1r), 0.0)
        inter = iw * ih
        iou = inter / (ac + arear - inter + 1e-6)
        ii = jax.lax.broadcasted_iota(jnp.int32, (_STRIP, _PRE), 0) + r0
        jj = jax.lax.broadcasted_iota(jnp.int32, (_STRIP, _PRE), 1)
        s_scr[r0:r0 + _STRIP, :] = jnp.where(
            (iou > _IOU_TH) & (jj > ii), 1.0, 0.0).astype(jnp.bfloat16)

    S = s_scr[...]

    bf16 = jnp.bfloat16
    dot01 = functools.partial(jnp.dot, preferred_element_type=f32)

    valid_f = valid.astype(f32)

    def cond(st):
        _, _, undecided = st
        return undecided > 0.0

    def step(st):
        kept, dead, _ = st
        alive = valid_f * (1.0 - dead)
        threat = dot01(alive.astype(bf16), S)
        kept_n = jnp.where(threat < 0.5, valid_f, 0.0)
        dead_n = jnp.where(
            dot01(kept_n.astype(bf16), S) > 0.5, 1.0, 0.0)
        und = jnp.sum(valid_f * (1.0 - kept_n) * (1.0 - dead_n))
        return kept_n, dead_n, und

    zeros_f = jnp.zeros((1, _PRE), dtype=f32)
    und0 = jnp.sum(valid_f)
    kept_f, _, _ = jax.lax.while_loop(
        cond, step, (zeros_f, zeros_f, und0))

    rank = kept_f
    d = 1
    while d < _PRE:
        rank = rank + jnp.concatenate(
            [jnp.zeros((1, d), f32), rank[:, :_PRE - d]], axis=1)
        d *= 2
    slot = rank - 1.0

    rows = jax.lax.broadcasted_iota(jnp.int32, (_POST_PAD, 1), 0)
    slot_i = slot.astype(jnp.int32)
    O = jnp.where((rows == slot_i) & (kept_f > 0.5), 1.0, 0.0).astype(f32)

    sel = jnp.dot(O, TC, preferred_element_type=f32,
                  precision=jax.lax.Precision.HIGHEST)
    sel_ref[0] = sel

    kcnt = jnp.sum(kept_f).astype(jnp.int32)
    maskc = jnp.where(rows < kcnt, 1.0, 0.0).astype(f32)
    cmax = jnp.max(sel[:, 8:11], axis=1, keepdims=True)
    small = jnp.concatenate(
        [maskc, cmax * maskc, jnp.zeros((_POST_PAD, 6), f32)], axis=1)
    small_ref[0] = small


def kernel(batch_box_preds_tea, batch_box_preds_stu, batch_cls_preds_stu,
           cls_preds):
    f32 = jnp.float32
    dense_cls = cls_preds.reshape(_B, -1, _C)

    scores = jnp.max(batch_cls_preds_stu, axis=2)
    masked = jnp.where(scores > _SCORE_TH, scores, -jnp.inf)
    top_scores, order = jax.lax.top_k(masked, _PRE)

    oix = order[:, :, None]
    cand_stu = jnp.take_along_axis(batch_box_preds_stu, oix, axis=1)
    cand_tea = jnp.take_along_axis(batch_box_preds_tea, oix, axis=1)
    cand_cls = jnp.take_along_axis(dense_cls, oix, axis=1)

    a_in = jnp.concatenate([cand_stu, top_scores[:, :, None]], axis=2)
    bt_in = jnp.swapaxes(a_in, 1, 2)
    tc_in = jnp.concatenate(
        [cand_tea, jnp.zeros((_B, _PRE, 1), f32),
         cand_cls, jnp.zeros((_B, _PRE, 5), f32)], axis=2)

    sel, small = pl.pallas_call(
        _nms_body,
        grid=(_B,),
        in_specs=[
            pl.BlockSpec((1, _PRE, 8), lambda b: (b, 0, 0)),
            pl.BlockSpec((1, 8, _PRE), lambda b: (b, 0, 0)),
            pl.BlockSpec((1, _PRE, 16), lambda b: (b, 0, 0)),
        ],
        out_specs=[
            pl.BlockSpec((1, _POST_PAD, 16), lambda b: (b, 0, 0)),
            pl.BlockSpec((1, _POST_PAD, 8), lambda b: (b, 0, 0)),
        ],
        out_shape=[
            jax.ShapeDtypeStruct((_B, _POST_PAD, 16), f32),
            jax.ShapeDtypeStruct((_B, _POST_PAD, 8), f32),
        ],
        scratch_shapes=[pltpu.VMEM((_PRE, _PRE), jnp.bfloat16)],
    )(a_in, bt_in, tc_in)

    rois = sel[:, :_POST, 0:7]
    cls_sel = sel[:, :_POST, 8:11]
    mask = small[:, :_POST, 0]
    kd = small[:, :_POST, 1].reshape(-1, 1)
    return rois, cls_sel, mask, kd


# speedup vs baseline: 73.0121x; 1.2584x over previous
import functools

import jax
import jax.numpy as jnp
from jax.experimental import pallas as pl
from jax.experimental.pallas import tpu as pltpu

_B, _N, _C = 4, 20000, 3
_PRE = 2048
_POST = 500
_POST_PAD = 512
_IOU_TH = 0.7
_SCORE_TH = 0.1
_STRIP = 256


def _nms_body(a_ref, bt_ref, sel_ref, small_ref, s_scr):
    A = a_ref[0]
    Bt = bt_ref[0]

    f32 = jnp.float32

    x1c = A[:, 0:1] - A[:, 3:4] * 0.5
    x2c = A[:, 0:1] + A[:, 3:4] * 0.5
    y1c = A[:, 1:2] - A[:, 4:5] * 0.5
    y2c = A[:, 1:2] + A[:, 4:5] * 0.5
    areac = (x2c - x1c) * (y2c - y1c)

    x1r = Bt[0:1, :] - Bt[3:4, :] * 0.5
    x2r = Bt[0:1, :] + Bt[3:4, :] * 0.5
    y1r = Bt[1:2, :] - Bt[4:5, :] * 0.5
    y2r = Bt[1:2, :] + Bt[4:5, :] * 0.5
    arear = (x2r - x1r) * (y2r - y1r)

    ts_row = Bt[7:8, :]
    valid = ts_row > -jnp.inf

    s_scr[...] = jnp.zeros((_PRE, _PRE), jnp.bfloat16)
    for k in range(_PRE // _STRIP):
        r0 = k * _STRIP
        w = _PRE - r0
        xc1 = x1c[r0:r0 + _STRIP]
        xc2 = x2c[r0:r0 + _STRIP]
        yc1 = y1c[r0:r0 + _STRIP]
        yc2 = y2c[r0:r0 + _STRIP]
        ac = areac[r0:r0 + _STRIP]
        iw = jnp.maximum(
            jnp.minimum(xc2, x2r[:, r0:]) - jnp.maximum(xc1, x1r[:, r0:]),
            0.0)
        ih = jnp.maximum(
            jnp.minimum(yc2, y2r[:, r0:]) - jnp.maximum(yc1, y1r[:, r0:]),
            0.0)
        inter = iw * ih
        iou = inter / (ac + arear[:, r0:] - inter + 1e-6)
        ii = jax.lax.broadcasted_iota(jnp.int32, (_STRIP, w), 0) + r0
        jj = jax.lax.broadcasted_iota(jnp.int32, (_STRIP, w), 1) + r0
        s_scr[r0:r0 + _STRIP, r0:] = jnp.where(
            (iou > _IOU_TH) & (jj > ii), 1.0, 0.0).astype(jnp.bfloat16)

    S = s_scr[...]

    bf16 = jnp.bfloat16
    dot01 = functools.partial(jnp.dot, preferred_element_type=f32)

    valid_f = valid.astype(f32)

    def cond(st):
        _, _, undecided = st
        return undecided > 0.0

    def step(st):
        kept, dead, _ = st
        alive = valid_f * (1.0 - dead)
        threat = dot01(alive.astype(bf16), S)
        kept_n = jnp.where(threat < 0.5, valid_f, 0.0)
        dead_n = jnp.where(
            dot01(kept_n.astype(bf16), S) > 0.5, 1.0, 0.0)
        und = jnp.sum(valid_f * (1.0 - kept_n) * (1.0 - dead_n))
        return kept_n, dead_n, und

    zeros_f = jnp.zeros((1, _PRE), dtype=f32)
    und0 = jnp.sum(valid_f)
    kept_f, _, _ = jax.lax.while_loop(
        cond, step, (zeros_f, zeros_f, und0))

    rank = kept_f
    d = 1
    while d < _PRE:
        rank = rank + jnp.concatenate(
            [jnp.zeros((1, d), f32), rank[:, :_PRE - d]], axis=1)
        d *= 2
    slot = rank - 1.0

    rows = jax.lax.broadcasted_iota(jnp.int32, (_POST_PAD, 1), 0)
    slot_i = slot.astype(jnp.int32)
    O = jnp.where((rows == slot_i) & (kept_f > 0.5), 1.0, 0.0).astype(f32)

    sel = jnp.dot(O, A[:, 8:18], preferred_element_type=f32,
                  precision=jax.lax.Precision.HIGHEST)
    sel_ref[0] = jnp.concatenate(
        [sel, jnp.zeros((_POST_PAD, 6), f32)], axis=1)

    kcnt = jnp.sum(kept_f).astype(jnp.int32)
    maskc = jnp.where(rows < kcnt, 1.0, 0.0).astype(f32)
    cmax = jnp.max(sel[:, 7:10], axis=1, keepdims=True)
    small = jnp.concatenate(
        [maskc, cmax * maskc, jnp.zeros((_POST_PAD, 6), f32)], axis=1)
    small_ref[0] = small


def kernel(batch_box_preds_tea, batch_box_preds_stu, batch_cls_preds_stu,
           cls_preds):
    f32 = jnp.float32
    dense_cls = cls_preds.reshape(_B, -1, _C)

    scores = jnp.max(batch_cls_preds_stu, axis=2)
    masked = jnp.where(scores > _SCORE_TH, scores, -jnp.inf)
    top_scores, order = jax.lax.top_k(masked, _PRE)

    X = jnp.concatenate(
        [batch_box_preds_stu, masked[:, :, None],
         batch_box_preds_tea, dense_cls], axis=2)
    cand = jnp.take_along_axis(X, order[:, :, None], axis=1)
    bt_in = jnp.swapaxes(cand[:, :, 0:8], 1, 2)

    sel, small = pl.pallas_call(
        _nms_body,
        grid=(_B,),
        in_specs=[
            pl.BlockSpec((1, _PRE, 18), lambda b: (b, 0, 0)),
            pl.BlockSpec((1, 8, _PRE), lambda b: (b, 0, 0)),
        ],
        out_specs=[
            pl.BlockSpec((1, _POST_PAD, 16), lambda b: (b, 0, 0)),
            pl.BlockSpec((1, _POST_PAD, 8), lambda b: (b, 0, 0)),
        ],
        out_shape=[
            jax.ShapeDtypeStruct((_B, _POST_PAD, 16), f32),
            jax.ShapeDtypeStruct((_B, _POST_PAD, 8), f32),
        ],
        scratch_shapes=[pltpu.VMEM((_PRE, _PRE), jnp.bfloat16)],
    )(cand, bt_in)

    rois = sel[:, :_POST, 0:7]
    cls_sel = sel[:, :_POST, 7:10]
    mask = small[:, :_POST, 0]
    kd = small[:, :_POST, 1].reshape(-1, 1)
    return rois, cls_sel, mask, kd
